# baseline (speedup 1.0000x reference)
import functools

import jax
import jax.numpy as jnp
from jax import lax
from jax.experimental import pallas as pl
from jax.experimental.pallas import tpu as pltpu
from jax.experimental.pallas import tpu_sc as plsc

T = 60
B = 4096
C = 5
D = 20
H4 = 4 * D
N0 = 3190
N0P = 3200
E0 = 51040
EC = 512
NEC = 100
F1 = 15
BB_ENC = 4096
BB_KNN = 512
NEG = float("-inf")


DP = 24
CP = 8


def _encode_body(xT_ref, embp_ref, wp_ref, biasp_ref, w1p_ref, b1_ref,
                 w2c_ref, b2_ref, xfT_ref, seq, esc):
    bb = xT_ref.shape[2]
    embp = embp_ref[...]
    w1p = w1p_ref[...]
    b1 = b1_ref[...]
    w2c = w2c_ref[...]
    b2 = b2_ref[...]
    wps = [wp_ref[l] for l in range(4)]
    biases = [biasp_ref[l] for l in range(4)]

    def cell(l, xin, h, c):
        g = (jnp.dot(wps[l], jnp.concatenate([xin, h], axis=0),
                     preferred_element_type=jnp.float32) + biases[l])
        ig = 0.5 + 0.5 * jnp.tanh(0.5 * g[0 * DP:1 * DP])
        fg = 0.5 + 0.5 * jnp.tanh(0.5 * g[1 * DP:2 * DP])
        gg = jnp.tanh(g[2 * DP:3 * DP])
        og = 0.5 + 0.5 * jnp.tanh(0.5 * g[3 * DP:4 * DP])
        cn = fg * c + ig * gg
        hn = og * jnp.tanh(cn)
        return hn, cn

    z = jnp.zeros((DP, bb), jnp.float32)
    hs = [z, z, z, z]
    cs = [z, z, z, z]
    for i in range(3):
        new_hs = list(hs)
        new_cs = list(cs)
        for l in range(i + 1):
            xin = (jax.nn.relu(jnp.dot(embp, xT_ref[i],
                                       preferred_element_type=jnp.float32))
                   if l == 0 else hs[l - 1])
            new_hs[l], new_cs[l] = cell(l, xin, hs[l], cs[l])
        hs, cs = new_hs, new_cs

    def wave(i, carry):
        hs, cs = carry
        tin = jnp.minimum(i, T - 1)
        x0in = jax.nn.relu(jnp.dot(embp, xT_ref[tin],
                                   preferred_element_type=jnp.float32))
        xins = [x0in, hs[0], hs[1], hs[2]]
        new = [cell(l, xins[l], hs[l], cs[l]) for l in range(4)]
        h3 = new[3][0]
        seq[i - 3] = h3
        e1 = jax.nn.relu(jnp.dot(w1p, h3,
                                 preferred_element_type=jnp.float32) + b1)
        esc[i - 3] = jnp.sum(e1 * w2c, axis=0, keepdims=True) + b2
        return ([n[0] for n in new], [n[1] for n in new])

    lax.fori_loop(3, T + 3, wave, (hs, cs))

    en = esc[...]
    m = jnp.max(en, axis=0, keepdims=True)
    w = jnp.exp(en - m)
    esc[...] = w / jnp.sum(w, axis=0, keepdims=True)

    TC_ = 6

    def wsum_chunk(i, acc):
        blk = seq[pl.ds(i * TC_, TC_)] * esc[pl.ds(i * TC_, TC_)]
        return acc + jnp.sum(blk, axis=0)

    xf = lax.fori_loop(0, T // TC_, wsum_chunk,
                       jnp.zeros((DP, bb), jnp.float32))
    xfT_ref[...] = xf


def _encode(xT, embp, wp, biasp, w1p, b1, w2c, b2):
    grid = (B // BB_ENC,)
    return pl.pallas_call(
        _encode_body,
        grid=grid,
        in_specs=[
            pl.BlockSpec((T, CP, BB_ENC), lambda i: (0, 0, i)),
            pl.BlockSpec((DP, CP), lambda i: (0, 0)),
            pl.BlockSpec((4, 4 * DP, 2 * DP), lambda i: (0, 0, 0)),
            pl.BlockSpec((4, 4 * DP, 1), lambda i: (0, 0, 0)),
            pl.BlockSpec((64, DP), lambda i: (0, 0)),
            pl.BlockSpec((64, 1), lambda i: (0, 0)),
            pl.BlockSpec((64, 1), lambda i: (0, 0)),
            pl.BlockSpec((1, 1), lambda i: (0, 0)),
        ],
        out_specs=pl.BlockSpec((DP, BB_ENC), lambda i: (0, i)),
        out_shape=jax.ShapeDtypeStruct((DP, B), jnp.float32),
        scratch_shapes=[
            pltpu.VMEM((T, DP, BB_ENC), jnp.float32),
            pltpu.VMEM((T, 1, BB_ENC), jnp.float32),
        ],
    )(xT, embp, wp, biasp, w1p, b1, w2c, b2)


SC_NC = 2
SC_NS = 16
SC_NW = SC_NC * SC_NS
SC_CH = 128
SC_IT = 13
EPW = SC_CH * SC_IT
EPAD = EPW * SC_NW
ROWS_PER_SUB = N0P // SC_NS


def _sc_segsum_body(x0_hbm, src_hbm, dst_hbm, zeros_hbm, out_hbm,
                    src_all, dst_all, rows0, rows1, shared, table_sh,
                    sem0, sem1):
    c = lax.axis_index("c")
    s = lax.axis_index("s")
    wid = s * SC_NC + c

    @pl.when(s == 0)
    def _zero():
        pltpu.sync_copy(zeros_hbm, shared)

    @pl.when(s == 1)
    def _stage_table():
        pltpu.sync_copy(x0_hbm, table_sh)

    plsc.subcore_barrier()
    pltpu.sync_copy(src_hbm.at[wid], src_all)
    pltpu.sync_copy(dst_hbm.at[wid], dst_all)

    rows = (rows0, rows1)
    sems = (sem0, sem1)
    cps = [None] * SC_IT
    cps[0] = pltpu.async_copy(table_sh.at[src_all.at[0]], rows0, sem0)
    for k in range(SC_IT):
        if k + 1 < SC_IT:
            cps[k + 1] = pltpu.async_copy(table_sh.at[src_all.at[k + 1]],
                                          rows[(k + 1) % 2],
                                          sems[(k + 1) % 2])
        cps[k].wait()
        pltpu.sync_copy(rows[k % 2], shared.at[dst_all.at[k]], add=True)

    plsc.subcore_barrier()
    pltpu.sync_copy(shared.at[pl.ds(s * ROWS_PER_SUB, ROWS_PER_SUB)],
                    out_hbm.at[c, pl.ds(s * ROWS_PER_SUB, ROWS_PER_SUB)])


def _sc_segsum(x0g, src3, dst3, zeros):
    mesh = plsc.VectorSubcoreMesh(core_axis_name="c", subcore_axis_name="s")
    fn = pl.kernel(
        _sc_segsum_body, mesh=mesh,
        out_type=jax.ShapeDtypeStruct((SC_NC, N0P, 128), jnp.float32),
        scratch_types=[
            pltpu.VMEM((SC_IT, SC_CH), jnp.int32),
            pltpu.VMEM((SC_IT, SC_CH), jnp.int32),
            pltpu.VMEM((SC_CH, 128), jnp.float32),
            pltpu.VMEM((SC_CH, 128), jnp.float32),
            pltpu.VMEM_SHARED((N0P, 128), jnp.float32),
            pltpu.VMEM_SHARED((N0P, 128), jnp.float32),
            pltpu.SemaphoreType.DMA,
            pltpu.SemaphoreType.DMA,
        ],
    )
    return fn(x0g, src3, dst3, zeros)


def _h0_fin_body(agg2_ref, x0p_ref, c1WlT_ref, c1bl_ref, c1WrT_ref, h0_ref):
    a = agg2_ref[0] + agg2_ref[1]
    agg = a[:, :D] / jnp.maximum(a[:, D:D + 1], jnp.float32(1))
    h0 = (jnp.dot(agg, c1WlT_ref[...], preferred_element_type=jnp.float32)
          + c1bl_ref[...]
          + jnp.dot(x0p_ref[:, :D], c1WrT_ref[...],
                    preferred_element_type=jnp.float32))
    h0_ref[...] = jax.nn.relu(h0)


def _h0_finalize(agg2, x0p, c1WlT, c1bl, c1WrT):
    return pl.pallas_call(
        _h0_fin_body,
        out_shape=jax.ShapeDtypeStruct((N0P, F1), jnp.float32),
    )(agg2, x0p, c1WlT, c1bl, c1WrT)


def _h0_body(src_ref, dst_ref, x0p_ref, c1WlT_ref, c1bl_ref, c1WrT_ref,
             h0_ref, acc):
    i = pl.program_id(0)

    @pl.when(i == 0)
    def _init():
        acc[...] = jnp.zeros_like(acc)

    srcs_col = src_ref[0]
    dsts_row = dst_ref[0]
    valid_row = (i * EC
                 + lax.broadcasted_iota(jnp.int32, (1, EC), 1)) < E0
    valid_col = (i * EC
                 + lax.broadcasted_iota(jnp.int32, (EC, 1), 0)) < E0

    row_io = lax.broadcasted_iota(jnp.int32, (N0P, EC), 0)
    dmat = jnp.where((row_io == dsts_row) & valid_row,
                     jnp.float32(1), jnp.float32(0))
    col_io = lax.broadcasted_iota(jnp.int32, (EC, N0P), 1)
    smat = jnp.where((col_io == srcs_col) & valid_col,
                     jnp.float32(1), jnp.float32(0))

    msg = jnp.dot(smat, x0p_ref[...], preferred_element_type=jnp.float32)
    acc[...] += jnp.dot(dmat, msg, preferred_element_type=jnp.float32)

    @pl.when(i == NEC - 1)
    def _finalize():
        a = acc[...]
        agg = a[:, :D] / jnp.maximum(a[:, D:D + 1], jnp.float32(1))
        h0 = (jnp.dot(agg, c1WlT_ref[...], preferred_element_type=jnp.float32)
              + c1bl_ref[...]
              + jnp.dot(x0p_ref[:, :D], c1WrT_ref[...],
                        preferred_element_type=jnp.float32))
        h0_ref[...] = jax.nn.relu(h0)


def _h0_compute(src3, dst3, x0p, c1WlT, c1bl, c1WrT):
    return pl.pallas_call(
        _h0_body,
        grid=(NEC,),
        in_specs=[
            pl.BlockSpec((1, EC, 1), lambda i: (i, 0, 0)),
            pl.BlockSpec((1, 1, EC), lambda i: (i, 0, 0)),
            pl.BlockSpec((N0P, 32), lambda i: (0, 0)),
            pl.BlockSpec((D, F1), lambda i: (0, 0)),
            pl.BlockSpec((1, F1), lambda i: (0, 0)),
            pl.BlockSpec((D, F1), lambda i: (0, 0)),
        ],
        out_specs=pl.BlockSpec((N0P, F1), lambda i: (0, 0)),
        out_shape=jax.ShapeDtypeStruct((N0P, F1), jnp.float32),
        scratch_shapes=[pltpu.VMEM((N0P, 32), jnp.float32)],
    )(src3, dst3, x0p, c1WlT, c1bl, c1WrT)


def _knn_body(xf_ref, x0pT_ref, agg2_ref, x0p_ref, c1WlT_ref, c1bl_ref,
              c1WrT_ref, c2WlT_ref, c2bl_ref, c2WrT_ref, linWT_ref,
              linb_ref, out_ref, h0_ref):
    bb = xf_ref.shape[0]

    @pl.when(pl.program_id(0) == 0)
    def _h0():
        a = agg2_ref[0] + agg2_ref[1]
        agg = a[:, :D] / jnp.maximum(a[:, D:D + 1], jnp.float32(1))
        h0 = (jnp.dot(agg, c1WlT_ref[...], preferred_element_type=jnp.float32)
              + c1bl_ref[...]
              + jnp.dot(x0p_ref[:, :D], c1WrT_ref[...],
                        preferred_element_type=jnp.float32))
        h0_ref[...] = jax.nn.relu(h0)

    xf = xf_ref[...]
    x0T = x0pT_ref[:D]

    nq = jnp.sqrt(jnp.sum(xf * xf, axis=1, keepdims=True))
    n0 = jnp.sqrt(jnp.sum(x0T * x0T, axis=0, keepdims=True))
    n0 = jnp.maximum(n0, jnp.float32(1e-30))

    a = jnp.dot(xf, x0T, preferred_element_type=jnp.float32)
    sim = a / nq / n0
    col_io = lax.broadcasted_iota(jnp.int32, (bb, N0P), 1)
    sim = jnp.where(col_io < N0, sim, NEG)

    run = sim
    amid = jnp.zeros((bb, N0P), jnp.float32)
    oh0 = None
    oh5 = None
    v0 = None
    for k in range(6):
        m = jnp.max(run, axis=1, keepdims=True)
        oh = (run == m)
        ohf = oh.astype(jnp.float32)
        if k == 0:
            v0 = m
            oh0 = ohf
        elif k == 5:
            oh5 = ohf
        else:
            amid = amid + ohf
        run = jnp.where(oh, NEG, run)

    cond = v0 == jnp.float32(1.0)
    amat = amid + jnp.where(cond, oh5, oh0)

    aggq = jnp.dot(amat, h0_ref[...],
                   preferred_element_type=jnp.float32) * jnp.float32(0.2)
    hq = jax.nn.relu(jnp.dot(xf, c1WrT_ref[...],
                             preferred_element_type=jnp.float32)
                     + c1bl_ref[...])
    outq = (jnp.dot(aggq, c2WlT_ref[...], preferred_element_type=jnp.float32)
            + c2bl_ref[...]
            + jnp.dot(hq, c2WrT_ref[...], preferred_element_type=jnp.float32))
    logits = (jnp.dot(outq, linWT_ref[...], preferred_element_type=jnp.float32)
              + linb_ref[...])
    m = jnp.max(logits, axis=1, keepdims=True)
    e = jnp.exp(logits - m)
    out_ref[...] = e / jnp.sum(e, axis=1, keepdims=True)


def _knn_head(xf, x0pT, agg2, x0p, c1WlT, c1bl, c1WrT, c2WlT, c2bl, c2WrT,
              linWT, linb):
    grid = (B // BB_KNN,)
    return pl.pallas_call(
        _knn_body,
        grid=grid,
        in_specs=[
            pl.BlockSpec((BB_KNN, D), lambda i: (i, 0)),
            pl.BlockSpec((32, N0P), lambda i: (0, 0)),
            pl.BlockSpec((SC_NC, N0P, 128), lambda i: (0, 0, 0)),
            pl.BlockSpec((N0P, 32), lambda i: (0, 0)),
            pl.BlockSpec((D, F1), lambda i: (0, 0)),
            pl.BlockSpec((1, F1), lambda i: (0, 0)),
            pl.BlockSpec((D, F1), lambda i: (0, 0)),
            pl.BlockSpec((F1, D), lambda i: (0, 0)),
            pl.BlockSpec((1, D), lambda i: (0, 0)),
            pl.BlockSpec((F1, D), lambda i: (0, 0)),
            pl.BlockSpec((D, 3), lambda i: (0, 0)),
            pl.BlockSpec((1, 3), lambda i: (0, 0)),
        ],
        out_specs=pl.BlockSpec((BB_KNN, 3), lambda i: (i, 0)),
        out_shape=jax.ShapeDtypeStruct((B, 3), jnp.float32),
        scratch_shapes=[pltpu.VMEM((N0P, F1), jnp.float32)],
    )(xf, x0pT, agg2, x0p, c1WlT, c1bl, c1WrT, c2WlT, c2bl, c2WrT, linWT,
      linb)


def kernel(x, x_0, edge_0, emb, lstm_Wih, lstm_Whh, lstm_bih, lstm_bhh,
           att_W1, att_b1, att_W2, att_b2,
           c1_Wl, c1_bl, c1_Wr, c2_Wl, c2_bl, c2_Wr, lin_W, lin_b):
    f32 = jnp.float32
    wih4 = lstm_Wih.astype(f32).reshape(4, 4, D, D)
    whh4 = lstm_Whh.astype(f32).reshape(4, 4, D, D)
    wih4 = jnp.pad(wih4, ((0, 0), (0, 0), (0, DP - D), (0, DP - D)))
    whh4 = jnp.pad(whh4, ((0, 0), (0, 0), (0, DP - D), (0, DP - D)))
    wp = jnp.concatenate([wih4, whh4], axis=3).reshape(4, 4 * DP, 2 * DP)
    biasp = jnp.pad((lstm_bih + lstm_bhh).astype(f32).reshape(4, 4, D),
                    ((0, 0), (0, 0), (0, DP - D))).reshape(4, 4 * DP, 1)
    embp = jnp.pad(emb.T.astype(f32), ((0, DP - D), (0, CP - C)))
    w1p = jnp.pad(att_W1.astype(f32), ((0, 0), (0, DP - D)))
    b1 = att_b1.reshape(64, 1).astype(f32)
    w2c = att_W2.reshape(64, 1).astype(f32)
    b2 = att_b2.reshape(1, 1).astype(f32)
    xT = jnp.pad(jnp.transpose(x.astype(f32), (0, 2, 1)),
                 ((0, 0), (0, CP - C), (0, 0)))

    x0p = jnp.zeros((N0P, 32), f32)
    x0p = x0p.at[:N0, :D].set(x_0.astype(f32))
    x0p = x0p.at[:N0, D].set(f32(1))

    e = edge_0.astype(jnp.int32)
    src_pad = jnp.pad(e[0], (0, EPAD - E0)).reshape(SC_NW, SC_IT, SC_CH)
    dst_pad = jnp.pad(e[1], (0, EPAD - E0),
                      constant_values=N0P - 1).reshape(SC_NW, SC_IT, SC_CH)
    zeros = jnp.zeros((N0P, 128), f32)
    x0g = jnp.pad(x0p, ((0, 0), (0, 96)))
    x0pT = x0p.T

    c1WlT = c1_Wl.T.astype(f32)
    c1bl = c1_bl.reshape(1, F1).astype(f32)
    c1WrT = c1_Wr.T.astype(f32)
    c2WlT = c2_Wl.T.astype(f32)
    c2bl = c2_bl.reshape(1, D).astype(f32)
    c2WrT = c2_Wr.T.astype(f32)
    linWT = lin_W.T.astype(f32)
    linb = lin_b.reshape(1, 3).astype(f32)

    xfT = _encode(xT, embp, wp, biasp, w1p, b1, w2c, b2)
    xf = xfT[:D].T
    agg2 = _sc_segsum(x0g, src_pad, dst_pad, zeros)
    return _knn_head(xf, x0pT, agg2, x0p, c1WlT, c1bl, c1WrT, c2WlT, c2bl,
                     c2WrT, linWT, linb)


# speedup vs baseline: 1.0248x; 1.0248x over previous
import functools

import jax
import jax.numpy as jnp
from jax import lax
from jax.experimental import pallas as pl
from jax.experimental.pallas import tpu as pltpu
from jax.experimental.pallas import tpu_sc as plsc

T = 60
B = 4096
C = 5
D = 20
H4 = 4 * D
N0 = 3190
N0P = 3200
E0 = 51040
EC = 512
NEC = 100
F1 = 15
BB_ENC = 4096
BB_KNN = 512
NEG = float("-inf")


DP = 24
CP = 8


def _encode_body(xT_ref, embp_ref, wp_ref, biasp_ref, w1p_ref, b1_ref,
                 w2c_ref, b2_ref, xfT_ref, seq, esc):
    bb = xT_ref.shape[2]
    embp = embp_ref[...]
    w1p = w1p_ref[...]
    b1 = b1_ref[...]
    w2c = w2c_ref[...]
    b2 = b2_ref[...]
    wps = [wp_ref[l] for l in range(4)]
    biases = [biasp_ref[l] for l in range(4)]

    def cell(l, xin, h, c):
        g = (jnp.dot(wps[l], jnp.concatenate([xin, h], axis=0),
                     preferred_element_type=jnp.float32) + biases[l])
        ig = 0.5 + 0.5 * jnp.tanh(g[0 * DP:1 * DP])
        fg = 0.5 + 0.5 * jnp.tanh(g[1 * DP:2 * DP])
        gg = jnp.tanh(g[2 * DP:3 * DP])
        og = 0.5 + 0.5 * jnp.tanh(g[3 * DP:4 * DP])
        cn = fg * c + ig * gg
        hn = og * jnp.tanh(cn)
        return hn, cn

    z = jnp.zeros((DP, bb), jnp.float32)
    hs = [z, z, z, z]
    cs = [z, z, z, z]
    for i in range(3):
        new_hs = list(hs)
        new_cs = list(cs)
        for l in range(i + 1):
            xin = (jax.nn.relu(jnp.dot(embp, xT_ref[i],
                                       preferred_element_type=jnp.float32))
                   if l == 0 else hs[l - 1])
            new_hs[l], new_cs[l] = cell(l, xin, hs[l], cs[l])
        hs, cs = new_hs, new_cs

    def wave(i, carry):
        hs, cs = carry
        tin = jnp.minimum(i, T - 1)
        x0in = jax.nn.relu(jnp.dot(embp, xT_ref[tin],
                                   preferred_element_type=jnp.float32))
        xins = [x0in, hs[0], hs[1], hs[2]]
        new = [cell(l, xins[l], hs[l], cs[l]) for l in range(4)]
        h3 = new[3][0]
        seq[i - 3] = h3
        e1 = jax.nn.relu(jnp.dot(w1p, h3,
                                 preferred_element_type=jnp.float32) + b1)
        esc[i - 3] = jnp.sum(e1 * w2c, axis=0, keepdims=True) + b2
        return ([n[0] for n in new], [n[1] for n in new])

    lax.fori_loop(3, T + 3, wave, (hs, cs))

    en = esc[...]
    m = jnp.max(en, axis=0, keepdims=True)
    w = jnp.exp(en - m)
    esc[...] = w / jnp.sum(w, axis=0, keepdims=True)

    TC_ = 6

    def wsum_chunk(i, acc):
        blk = seq[pl.ds(i * TC_, TC_)] * esc[pl.ds(i * TC_, TC_)]
        return acc + jnp.sum(blk, axis=0)

    xf = lax.fori_loop(0, T // TC_, wsum_chunk,
                       jnp.zeros((DP, bb), jnp.float32))
    xfT_ref[...] = xf


def _encode(xT, embp, wp, biasp, w1p, b1, w2c, b2):
    grid = (B // BB_ENC,)
    return pl.pallas_call(
        _encode_body,
        grid=grid,
        in_specs=[
            pl.BlockSpec((T, CP, BB_ENC), lambda i: (0, 0, i)),
            pl.BlockSpec((DP, CP), lambda i: (0, 0)),
            pl.BlockSpec((4, 4 * DP, 2 * DP), lambda i: (0, 0, 0)),
            pl.BlockSpec((4, 4 * DP, 1), lambda i: (0, 0, 0)),
            pl.BlockSpec((64, DP), lambda i: (0, 0)),
            pl.BlockSpec((64, 1), lambda i: (0, 0)),
            pl.BlockSpec((64, 1), lambda i: (0, 0)),
            pl.BlockSpec((1, 1), lambda i: (0, 0)),
        ],
        out_specs=pl.BlockSpec((DP, BB_ENC), lambda i: (0, i)),
        out_shape=jax.ShapeDtypeStruct((DP, B), jnp.float32),
        scratch_shapes=[
            pltpu.VMEM((T, DP, BB_ENC), jnp.float32),
            pltpu.VMEM((T, 1, BB_ENC), jnp.float32),
        ],
    )(xT, embp, wp, biasp, w1p, b1, w2c, b2)


SC_NC = 2
SC_NS = 16
SC_NW = SC_NC * SC_NS
SC_CH = 128
SC_IT = 13
EPW = SC_CH * SC_IT
EPAD = EPW * SC_NW
ROWS_PER_SUB = N0P // SC_NS


def _sc_segsum_body(x0_hbm, src_hbm, dst_hbm, zeros_hbm, out_hbm,
                    src_all, dst_all, rows0, rows1, shared, table_sh,
                    sem0, sem1):
    c = lax.axis_index("c")
    s = lax.axis_index("s")
    wid = s * SC_NC + c

    @pl.when(s == 0)
    def _zero():
        pltpu.sync_copy(zeros_hbm, shared)

    @pl.when(s == 1)
    def _stage_table():
        pltpu.sync_copy(x0_hbm, table_sh)

    plsc.subcore_barrier()
    pltpu.sync_copy(src_hbm.at[wid], src_all)
    pltpu.sync_copy(dst_hbm.at[wid], dst_all)

    rows = (rows0, rows1)
    sems = (sem0, sem1)
    cps = [None] * SC_IT
    cps[0] = pltpu.async_copy(table_sh.at[src_all.at[0]], rows0, sem0)
    for k in range(SC_IT):
        if k + 1 < SC_IT:
            cps[k + 1] = pltpu.async_copy(table_sh.at[src_all.at[k + 1]],
                                          rows[(k + 1) % 2],
                                          sems[(k + 1) % 2])
        cps[k].wait()
        pltpu.sync_copy(rows[k % 2], shared.at[dst_all.at[k]], add=True)

    plsc.subcore_barrier()
    pltpu.sync_copy(shared.at[pl.ds(s * ROWS_PER_SUB, ROWS_PER_SUB)],
                    out_hbm.at[c, pl.ds(s * ROWS_PER_SUB, ROWS_PER_SUB)])


def _sc_segsum(x0g, src3, dst3, zeros):
    mesh = plsc.VectorSubcoreMesh(core_axis_name="c", subcore_axis_name="s")
    fn = pl.kernel(
        _sc_segsum_body, mesh=mesh,
        out_type=jax.ShapeDtypeStruct((SC_NC, N0P, 128), jnp.float32),
        scratch_types=[
            pltpu.VMEM((SC_IT, SC_CH), jnp.int32),
            pltpu.VMEM((SC_IT, SC_CH), jnp.int32),
            pltpu.VMEM((SC_CH, 128), jnp.float32),
            pltpu.VMEM((SC_CH, 128), jnp.float32),
            pltpu.VMEM_SHARED((N0P, 128), jnp.float32),
            pltpu.VMEM_SHARED((N0P, 128), jnp.float32),
            pltpu.SemaphoreType.DMA,
            pltpu.SemaphoreType.DMA,
        ],
    )
    return fn(x0g, src3, dst3, zeros)


def _knn_body(xf_ref, x0pT_ref, agg2_ref, x0p_ref, c1WlT_ref, c1bl_ref,
              c1WrT_ref, c2WlT_ref, c2bl_ref, c2WrT_ref, linWT_ref,
              linb_ref, out_ref, h0_ref):
    bb = xf_ref.shape[0]

    @pl.when(pl.program_id(0) == 0)
    def _h0():
        a = agg2_ref[0] + agg2_ref[1]
        agg = a[:, :D] / jnp.maximum(a[:, D:D + 1], jnp.float32(1))
        h0 = (jnp.dot(agg, c1WlT_ref[...], preferred_element_type=jnp.float32)
              + c1bl_ref[...]
              + jnp.dot(x0p_ref[:, :D], c1WrT_ref[...],
                        preferred_element_type=jnp.float32))
        h0_ref[...] = jax.nn.relu(h0)

    xf = xf_ref[...]
    x0T = x0pT_ref[:D]

    nq = jnp.sqrt(jnp.sum(xf * xf, axis=1, keepdims=True))
    n0 = jnp.sqrt(jnp.sum(x0T * x0T, axis=0, keepdims=True))
    n0 = jnp.maximum(n0, jnp.float32(1e-30))

    a = jnp.dot(xf, x0T, preferred_element_type=jnp.float32)
    sim = a / nq / n0
    col_io = lax.broadcasted_iota(jnp.int32, (bb, N0P), 1)
    sim = jnp.where(col_io < N0, sim, NEG)

    run = sim
    amid = jnp.zeros((bb, N0P), jnp.float32)
    oh0 = None
    oh5 = None
    v0 = None
    for k in range(6):
        m = jnp.max(run, axis=1, keepdims=True)
        oh = (run == m)
        ohf = oh.astype(jnp.float32)
        if k == 0:
            v0 = m
            oh0 = ohf
        elif k == 5:
            oh5 = ohf
        else:
            amid = amid + ohf
        run = jnp.where(oh, NEG, run)

    cond = v0 == jnp.float32(1.0)
    amat = amid + jnp.where(cond, oh5, oh0)

    aggq = jnp.dot(amat, h0_ref[...],
                   preferred_element_type=jnp.float32) * jnp.float32(0.2)
    hq = jax.nn.relu(jnp.dot(xf, c1WrT_ref[...],
                             preferred_element_type=jnp.float32)
                     + c1bl_ref[...])
    outq = (jnp.dot(aggq, c2WlT_ref[...], preferred_element_type=jnp.float32)
            + c2bl_ref[...]
            + jnp.dot(hq, c2WrT_ref[...], preferred_element_type=jnp.float32))
    logits = (jnp.dot(outq, linWT_ref[...], preferred_element_type=jnp.float32)
              + linb_ref[...])
    m = jnp.max(logits, axis=1, keepdims=True)
    e = jnp.exp(logits - m)
    out_ref[...] = e / jnp.sum(e, axis=1, keepdims=True)


def _knn_head(xf, x0pT, agg2, x0p, c1WlT, c1bl, c1WrT, c2WlT, c2bl, c2WrT,
              linWT, linb):
    grid = (B // BB_KNN,)
    return pl.pallas_call(
        _knn_body,
        grid=grid,
        in_specs=[
            pl.BlockSpec((BB_KNN, D), lambda i: (i, 0)),
            pl.BlockSpec((32, N0P), lambda i: (0, 0)),
            pl.BlockSpec((SC_NC, N0P, 128), lambda i: (0, 0, 0)),
            pl.BlockSpec((N0P, 32), lambda i: (0, 0)),
            pl.BlockSpec((D, F1), lambda i: (0, 0)),
            pl.BlockSpec((1, F1), lambda i: (0, 0)),
            pl.BlockSpec((D, F1), lambda i: (0, 0)),
            pl.BlockSpec((F1, D), lambda i: (0, 0)),
            pl.BlockSpec((1, D), lambda i: (0, 0)),
            pl.BlockSpec((F1, D), lambda i: (0, 0)),
            pl.BlockSpec((D, 3), lambda i: (0, 0)),
            pl.BlockSpec((1, 3), lambda i: (0, 0)),
        ],
        out_specs=pl.BlockSpec((BB_KNN, 3), lambda i: (i, 0)),
        out_shape=jax.ShapeDtypeStruct((B, 3), jnp.float32),
        scratch_shapes=[pltpu.VMEM((N0P, F1), jnp.float32)],
    )(xf, x0pT, agg2, x0p, c1WlT, c1bl, c1WrT, c2WlT, c2bl, c2WrT, linWT,
      linb)


def kernel(x, x_0, edge_0, emb, lstm_Wih, lstm_Whh, lstm_bih, lstm_bhh,
           att_W1, att_b1, att_W2, att_b2,
           c1_Wl, c1_bl, c1_Wr, c2_Wl, c2_bl, c2_Wr, lin_W, lin_b):
    f32 = jnp.float32
    gsc = jnp.array([0.5, 0.5, 1.0, 0.5], f32).reshape(1, 4, 1, 1)
    wih4 = lstm_Wih.astype(f32).reshape(4, 4, D, D) * gsc
    whh4 = lstm_Whh.astype(f32).reshape(4, 4, D, D) * gsc
    wih4 = jnp.pad(wih4, ((0, 0), (0, 0), (0, DP - D), (0, DP - D)))
    whh4 = jnp.pad(whh4, ((0, 0), (0, 0), (0, DP - D), (0, DP - D)))
    wp = jnp.concatenate([wih4, whh4], axis=3).reshape(4, 4 * DP, 2 * DP)
    biasp = jnp.pad((lstm_bih + lstm_bhh).astype(f32).reshape(4, 4, D)
                    * gsc.reshape(1, 4, 1),
                    ((0, 0), (0, 0), (0, DP - D))).reshape(4, 4 * DP, 1)
    embp = jnp.pad(emb.T.astype(f32), ((0, DP - D), (0, CP - C)))
    w1p = jnp.pad(att_W1.astype(f32), ((0, 0), (0, DP - D)))
    b1 = att_b1.reshape(64, 1).astype(f32)
    w2c = att_W2.reshape(64, 1).astype(f32)
    b2 = att_b2.reshape(1, 1).astype(f32)
    xT = jnp.pad(jnp.transpose(x.astype(f32), (0, 2, 1)),
                 ((0, 0), (0, CP - C), (0, 0)))

    x0p = jnp.zeros((N0P, 32), f32)
    x0p = x0p.at[:N0, :D].set(x_0.astype(f32))
    x0p = x0p.at[:N0, D].set(f32(1))

    e = edge_0.astype(jnp.int32)
    src_pad = jnp.pad(e[0], (0, EPAD - E0)).reshape(SC_NW, SC_IT, SC_CH)
    dst_pad = jnp.pad(e[1], (0, EPAD - E0),
                      constant_values=N0P - 1).reshape(SC_NW, SC_IT, SC_CH)
    zeros = jnp.zeros((N0P, 128), f32)
    x0g = jnp.pad(x0p, ((0, 0), (0, 96)))
    x0pT = x0p.T

    c1WlT = c1_Wl.T.astype(f32)
    c1bl = c1_bl.reshape(1, F1).astype(f32)
    c1WrT = c1_Wr.T.astype(f32)
    c2WlT = c2_Wl.T.astype(f32)
    c2bl = c2_bl.reshape(1, D).astype(f32)
    c2WrT = c2_Wr.T.astype(f32)
    linWT = lin_W.T.astype(f32)
    linb = lin_b.reshape(1, 3).astype(f32)

    xfT = _encode(xT, embp, wp, biasp, w1p, b1, w2c, b2)
    xf = xfT[:D].T
    agg2 = _sc_segsum(x0g, src_pad, dst_pad, zeros)
    return _knn_head(xf, x0pT, agg2, x0p, c1WlT, c1bl, c1WrT, c2WlT, c2bl,
                     c2WrT, linWT, linb)
